# baseline (speedup 1.0000x reference)
import jax
import jax.numpy as jnp
from jax import lax
from jax.experimental import pallas as pl
from jax.experimental.pallas import tpu as pltpu
from jax.experimental.pallas import tpu_sc as plsc

DIMS = 1024


def _pe_lookup_body(xi_hbm, yi_hbm, pe_x_hbm, pe_y_hbm, out_hbm,
                    idx_v, row_v, sem):
    wid = lax.axis_index("s") * 2 + lax.axis_index("c")

    @pl.when(wid == 0)
    def _():
        pltpu.sync_copy(xi_hbm, idx_v)
        pltpu.async_copy(pe_x_hbm.at[idx_v], row_v, sem).wait()
        pltpu.sync_copy(row_v, out_hbm.at[pl.ds(0, 1)])

    @pl.when(wid == 1)
    def _():
        pltpu.sync_copy(yi_hbm, idx_v)
        pltpu.async_copy(pe_y_hbm.at[idx_v], row_v, sem).wait()
        pltpu.sync_copy(row_v, out_hbm.at[pl.ds(1, 1)])


def kernel(xy_tensor, pe_x, pe_y):
    xi = xy_tensor[:, 0]
    yi = xy_tensor[:, 1]
    mesh = plsc.VectorSubcoreMesh(core_axis_name="c", subcore_axis_name="s")
    out = pl.kernel(
        _pe_lookup_body,
        out_type=jax.ShapeDtypeStruct((2, DIMS), jnp.float32),
        mesh=mesh,
        scratch_types=[
            pltpu.VMEM((1,), jnp.int32),
            pltpu.VMEM((1, DIMS), jnp.float32),
            pltpu.SemaphoreType.DMA,
        ],
    )(xi, yi, pe_x, pe_y)
    return out.reshape(1, 2 * DIMS)
